# Initial kernel scaffold; baseline (speedup 1.0000x reference)
import jax
import jax.numpy as jnp
from jax.experimental import pallas as pl


def kernel(x, weight):
    raise NotImplementedError("write your pallas kernel here")


# speedup vs baseline: 2.8066x; 2.8066x over previous
import jax
import jax.numpy as jnp
from jax.experimental import pallas as pl
from jax.experimental.pallas import tpu as pltpu


def _dwconv3x3_body(w_ref, x_ref, o_ref):
    c = jax.lax.rem(pl.program_id(0), w_ref.shape[0])
    xv = x_ref[0]
    h, w = xv.shape
    zrow = jnp.zeros((1, w), xv.dtype)
    zcol = jnp.zeros((h, 1), xv.dtype)
    rows = (
        jnp.concatenate([zrow, xv[:-1, :]], axis=0),
        xv,
        jnp.concatenate([xv[1:, :], zrow], axis=0),
    )
    acc = None
    for a in range(3):
        r = rows[a]
        for b in range(3):
            if b == 0:
                sh = jnp.concatenate([zcol, r[:, :-1]], axis=1)
            elif b == 1:
                sh = r
            else:
                sh = jnp.concatenate([r[:, 1:], zcol], axis=1)
            tap = w_ref[c, a * 3 + b] * sh
            acc = tap if acc is None else acc + tap
    o_ref[0] = acc


@jax.jit
def kernel(x, weight):
    n, c, h, w = x.shape
    kh, kw = weight.shape[-2], weight.shape[-1]
    xr = x.reshape(n * c, h, w)
    taps = weight.reshape(c, kh * kw)
    out = pl.pallas_call(
        _dwconv3x3_body,
        grid=(n * c,),
        in_specs=[
            pl.BlockSpec(memory_space=pltpu.SMEM),
            pl.BlockSpec((1, h, w), lambda i: (i, 0, 0)),
        ],
        out_specs=pl.BlockSpec((1, h, w), lambda i: (i, 0, 0)),
        out_shape=jax.ShapeDtypeStruct((n * c, h, w), x.dtype),
        compiler_params=pltpu.CompilerParams(
            dimension_semantics=("arbitrary",),
        ),
    )(taps, xr)
    return out.reshape(n, c, h, w)


# speedup vs baseline: 4.0016x; 1.4258x over previous
import jax
import jax.numpy as jnp
from jax.experimental import pallas as pl
from jax.experimental.pallas import tpu as pltpu


def _dwconv3x3_body(w_ref, x_ref, o_ref):
    c = jax.lax.rem(pl.program_id(0), w_ref.shape[0])
    xv = x_ref[0]
    h, w = xv.shape
    zrow = jnp.zeros((1, w), xv.dtype)
    zcol = jnp.zeros((h, 1), xv.dtype)
    cols = (
        jnp.concatenate([zcol, xv[:, :-1]], axis=1),
        xv,
        jnp.concatenate([xv[:, 1:], zcol], axis=1),
    )
    t = [
        w_ref[c, 3 * a] * cols[0]
        + w_ref[c, 3 * a + 1] * cols[1]
        + w_ref[c, 3 * a + 2] * cols[2]
        for a in range(3)
    ]
    o_ref[0] = (
        t[1]
        + jnp.concatenate([zrow, t[0][:-1, :]], axis=0)
        + jnp.concatenate([t[2][1:, :], zrow], axis=0)
    )


@jax.jit
def kernel(x, weight):
    n, c, h, w = x.shape
    kh, kw = weight.shape[-2], weight.shape[-1]
    xr = x.reshape(n * c, h, w)
    taps = weight.reshape(c, kh * kw)
    out = pl.pallas_call(
        _dwconv3x3_body,
        grid=(n * c,),
        in_specs=[
            pl.BlockSpec(memory_space=pltpu.SMEM),
            pl.BlockSpec((1, h, w), lambda i: (i, 0, 0)),
        ],
        out_specs=pl.BlockSpec((1, h, w), lambda i: (i, 0, 0)),
        out_shape=jax.ShapeDtypeStruct((n * c, h, w), x.dtype),
        compiler_params=pltpu.CompilerParams(
            dimension_semantics=("parallel",),
        ),
    )(taps, xr)
    return out.reshape(n, c, h, w)
